# baseline (speedup 1.0000x reference)
import functools

import jax
import jax.numpy as jnp
from jax import lax
from jax.experimental import pallas as pl
from jax.experimental.pallas import tpu as pltpu
from jax.experimental.pallas import tpu_sc as plsc

NC = 2
NS = 16
L = 16

_mesh = plsc.VectorSubcoreMesh(
    core_axis_name="c", subcore_axis_name="s", num_cores=NC, num_subcores=NS
)


def _softplus(r):
    return jnp.log(1.0 + jnp.exp(-jnp.abs(r))) + jnp.maximum(r, 0.0)


def _gate_body(ea_ref, gw1_ref, gb1_ref, gw2_ref, gb2_ref, o_ref):
    ea = ea_ref[...]
    h = lax.dot_general(ea, gw1_ref[...], (((1,), (1,)), ((), ())),
                        preferred_element_type=jnp.float32)
    h = jnp.maximum(h + gb1_ref[...], 0.0)
    s = jnp.sum(h * gw2_ref[...], axis=1) + gb2_ref[0, 0]
    o_ref[0, 0, :] = 1.0 / (1.0 + jnp.exp(-s))


def _xw_body(x_ref, wmu_ref, wrho_ref, epsw_ref, bmu_ref, brho_ref, epsb_ref,
             o_ref):
    w = wmu_ref[...] + _softplus(wrho_ref[...]) * epsw_ref[...]
    b = bmu_ref[...] + _softplus(brho_ref[...]) * epsb_ref[...]
    xw = lax.dot_general(x_ref[...], w, (((1,), (1,)), ((), ())),
                         preferred_element_type=jnp.float32) + b
    half = xw.shape[1] // 2
    o_ref[0] = xw[:, :half]
    o_ref[1] = xw[:, half:]


def _fin_body(ya_ref, yb_ref, xwa_ref, xwb_ref, deg_ref, o_ref):
    inv = 1.0 / deg_ref[0, 0, :]
    a = ya_ref[...] + xwa_ref[...] * inv[:, None]
    b = yb_ref[...] + xwb_ref[...] * inv[:, None]
    o_ref[...] = jnp.maximum(jnp.concatenate([a, b], axis=1), 0.0)


def _rsqrt_newton(p):
    i = plsc.bitcast(p, jnp.int32)
    i = jnp.int32(0x5F3759DF) - (i >> 1)
    y = plsc.bitcast(i, jnp.float32)
    for _ in range(3):
        y = y * (1.5 - 0.5 * p * y * y)
    return y


def kernel(x, edge_index, edge_attr, w_mu, w_rho, b_mu, b_rho, gW1, gb1, gW2,
           gb2, eps_w, eps_b):
    n, d = x.shape
    e = edge_index.shape[1]
    ef = edge_attr.shape[1]
    h_dim = gW1.shape[0]
    half = d // 2
    f32, i32 = jnp.float32, jnp.int32

    unit = NC * NS * 128
    e_pad = ((e + unit - 1) // unit) * unit
    ept = e_pad // (NC * NS)
    pad2 = 2 * e_pad
    spt = pad2 // 128 // NS
    c1 = 1024
    rows_t = ((n // NS + 7) // 8) * 8
    n_rows = NS * rows_t

    be = 20000
    g3 = pl.pallas_call(
        _gate_body,
        grid=(e // be,),
        in_specs=[
            pl.BlockSpec((be, ef), lambda i: (i, 0)),
            pl.BlockSpec((h_dim, ef), lambda i: (0, 0)),
            pl.BlockSpec((1, h_dim), lambda i: (0, 0)),
            pl.BlockSpec((1, h_dim), lambda i: (0, 0)),
            pl.BlockSpec((1, 1), lambda i: (0, 0)),
        ],
        out_specs=pl.BlockSpec((1, 1, be), lambda i: (i, 0, 0)),
        out_shape=jax.ShapeDtypeStruct((e // be, 1, be), f32),
    )(edge_attr, gW1, gb1.reshape(1, h_dim), gW2, gb2.reshape(1, 1))
    g = g3.reshape(e)

    bx = 1000
    xw2 = pl.pallas_call(
        _xw_body,
        grid=(n // bx,),
        in_specs=[
            pl.BlockSpec((bx, d), lambda i: (i, 0)),
            pl.BlockSpec((d, d), lambda i: (0, 0)),
            pl.BlockSpec((d, d), lambda i: (0, 0)),
            pl.BlockSpec((d, d), lambda i: (0, 0)),
            pl.BlockSpec((1, d), lambda i: (0, 0)),
            pl.BlockSpec((1, d), lambda i: (0, 0)),
            pl.BlockSpec((1, d), lambda i: (0, 0)),
        ],
        out_specs=pl.BlockSpec((2, bx, half), lambda i: (0, i, 0)),
        out_shape=jax.ShapeDtypeStruct((2, n, half), f32),
    )(x, w_mu, w_rho, eps_w, b_mu.reshape(1, d), b_rho.reshape(1, d),
      eps_b.reshape(1, d))
    xwflat = xw2.reshape(2 * n, half)

    zi = jnp.zeros((e_pad - e,), i32)
    zf = jnp.zeros((e_pad - e,), f32)
    ei0 = jnp.concatenate([edge_index[0].astype(i32), zi])
    ei1 = jnp.concatenate([edge_index[1].astype(i32), zi])
    gp = jnp.concatenate([g, zf])
    idxd = jnp.concatenate([ei0, ei1])
    idxs = jnp.concatenate([ei1, ei0])

    @functools.partial(
        pl.kernel,
        mesh=_mesh,
        compiler_params=pltpu.CompilerParams(needs_layout_passes=False),
        out_type=jax.ShapeDtypeStruct((NC, n), f32),
        scratch_types=[
            pltpu.VMEM((128,), i32),
            pltpu.VMEM((128,), f32),
            pltpu.VMEM((n,), f32),
            pltpu.VMEM_SHARED((n,), f32),
        ],
    )
    def _deg_kernel(ei0_ref, ei1_ref, g_ref, out_ref, idx_v, val_v, z_v, deg_sh):
        cid = lax.axis_index("c")
        sid = lax.axis_index("s")
        tid = sid * NC + cid

        @pl.when(sid == 0)
        def _():
            def zb(i, carry):
                z_v[pl.ds(i * L, L)] = jnp.zeros((L,), f32)
                return carry
            lax.fori_loop(0, n // L, zb, 0)
            pltpu.sync_copy(z_v, deg_sh)

        plsc.subcore_barrier()

        def chunk(k, carry):
            off = pl.multiple_of(tid * ept + k * 128, 128)
            pltpu.sync_copy(g_ref.at[pl.ds(off, 128)], val_v)
            pltpu.sync_copy(ei0_ref.at[pl.ds(off, 128)], idx_v)
            pltpu.sync_copy(val_v, deg_sh.at[idx_v], add=True)
            pltpu.sync_copy(ei1_ref.at[pl.ds(off, 128)], idx_v)
            pltpu.sync_copy(val_v, deg_sh.at[idx_v], add=True)
            return carry
        lax.fori_loop(0, ept // 128, chunk, 0)

        plsc.subcore_barrier()

        @pl.when(sid == 0)
        def _():
            pltpu.sync_copy(deg_sh, out_ref.at[cid])

    deg2 = _deg_kernel(ei0, ei1, gp)

    @functools.partial(
        pl.kernel,
        mesh=_mesh,
        compiler_params=pltpu.CompilerParams(needs_layout_passes=False),
        out_type=(
            jax.ShapeDtypeStruct((pad2,), f32),
            jax.ShapeDtypeStruct((n,), f32),
        ),
        scratch_types=[
            pltpu.VMEM((n,), f32),
            pltpu.VMEM((n,), f32),
            pltpu.VMEM((c1,), i32),
            pltpu.VMEM((c1,), i32),
            pltpu.VMEM((c1,), f32),
            pltpu.VMEM((c1,), f32),
        ],
    )
    def _coef_kernel(deg2_ref, ei0_ref, ei1_ref, g_ref, coef_ref, degout_ref,
                     deg_v, tmp_v, d_v, s_v, g_v, c_v):
        cid = lax.axis_index("c")
        sid = lax.axis_index("s")
        tid = sid * NC + cid
        pltpu.sync_copy(deg2_ref.at[0], deg_v)
        pltpu.sync_copy(deg2_ref.at[1], tmp_v)

        def addb(i, carry):
            sl = pl.ds(i * L, L)
            deg_v[sl] = deg_v[sl] + tmp_v[sl] + 1.0
            return carry
        lax.fori_loop(0, n // L, addb, 0)

        def chunk(j, carry):
            off = pl.multiple_of(tid * ept + j * c1, c1)
            pltpu.sync_copy(ei0_ref.at[pl.ds(off, c1)], d_v)
            pltpu.sync_copy(ei1_ref.at[pl.ds(off, c1)], s_v)
            pltpu.sync_copy(g_ref.at[pl.ds(off, c1)], g_v)

            def grp(i, carry2):
                sl = pl.ds(i * L, L)
                dd = plsc.load_gather(deg_v, [d_v[sl]])
                ds_ = plsc.load_gather(deg_v, [s_v[sl]])
                c_v[sl] = g_v[sl] * _rsqrt_newton(dd * ds_)
                return carry2
            lax.fori_loop(0, c1 // L, grp, 0)

            pltpu.sync_copy(c_v, coef_ref.at[pl.ds(off, c1)])
            off2 = pl.multiple_of(off + e_pad, c1)
            pltpu.sync_copy(c_v, coef_ref.at[pl.ds(off2, c1)])
            return carry
        lax.fori_loop(0, ept // c1, chunk, 0)

        @pl.when(jnp.logical_and(cid == 0, sid == 0))
        def _():
            pltpu.sync_copy(deg_v, degout_ref)

    coef, deg = _coef_kernel(deg2, ei0, ei1, gp)

    @functools.partial(
        pl.kernel,
        mesh=_mesh,
        compiler_params=pltpu.CompilerParams(needs_layout_passes=False),
        out_type=jax.ShapeDtypeStruct((2 * n_rows, half), f32),
        scratch_types=(
            [pltpu.VMEM((128, half), f32) for _ in range(2)]
            + [pltpu.VMEM((128,), i32) for _ in range(4)]
            + [
                pltpu.VMEM((4, 128), i32),
                pltpu.VMEM((4, 128), f32),
                pltpu.VMEM_SHARED((n_rows, half), f32),
                pltpu.SemaphoreType.DMA((4,)),
                pltpu.SemaphoreType.DMA((4,)),
                pltpu.SemaphoreType.DMA((2,)),
            ]
        ),
    )
    def _msg_kernel(tbl_ref, idxd_ref, idxs_ref, coef_ref,
                    yout_ref, data0, data1,
                    didx0, didx1, didx2, didx3, sidx_v, coef_v, y_sh,
                    sem_m, sem_g, sem_s):
        cid = lax.axis_index("c")
        sid = lax.axis_index("s")
        cbase = cid * n
        data = [data0, data1]
        didx = [didx0, didx1, didx2, didx3]

        def zrow(i, carry):
            for j in range(half // L):
                data0[i, pl.ds(j * L, L)] = jnp.zeros((L,), f32)
            return carry
        lax.fori_loop(0, 128, zrow, 0)

        nfull = rows_t // 128
        rem = rows_t - nfull * 128
        for q in range(nfull):
            pltpu.sync_copy(data0, y_sh.at[pl.ds(sid * rows_t + q * 128, 128)])
        if rem > 0:
            pltpu.sync_copy(data0.at[pl.ds(0, rem)],
                            y_sh.at[pl.ds(sid * rows_t + nfull * 128, rem)])

        plsc.subcore_barrier()

        if True:
            def _off(k):
                return pl.multiple_of((sid * spt + k) * 128, 128)

            def meta_start(k, mb):
                off = _off(k)
                pltpu.async_copy(idxs_ref.at[pl.ds(off, 128)],
                                 sidx_v.at[mb], sem_m.at[mb])
                pltpu.async_copy(idxd_ref.at[pl.ds(off, 128)],
                                 didx[mb], sem_m.at[mb])
                pltpu.async_copy(coef_ref.at[pl.ds(off, 128)],
                                 coef_v.at[mb], sem_m.at[mb])

            def meta_wait(k, mb):
                off = _off(k)
                pltpu.make_async_copy(idxs_ref.at[pl.ds(off, 128)],
                                      sidx_v.at[mb], sem_m.at[mb]).wait()
                pltpu.make_async_copy(idxd_ref.at[pl.ds(off, 128)],
                                      didx[mb], sem_m.at[mb]).wait()
                pltpu.make_async_copy(coef_ref.at[pl.ds(off, 128)],
                                      coef_v.at[mb], sem_m.at[mb]).wait()
                for j2 in range(128 // L):
                    sl = pl.ds(j2 * L, L)
                    sidx_v[mb, sl] = sidx_v[mb, sl] + cbase

            def gather_start(mb, db):
                pltpu.async_copy(tbl_ref.at[sidx_v.at[mb]], data[db],
                                 sem_g.at[mb])

            def gather_wait(mb, db):
                pltpu.make_async_copy(tbl_ref.at[sidx_v.at[mb]], data[db],
                                      sem_g.at[mb]).wait()

            def scatter_start(mb, db):
                pltpu.async_copy(data[db], y_sh.at[didx[mb]], sem_s.at[db],
                                 add=True)

            def scatter_wait(mb, db):
                pltpu.make_async_copy(data[db], y_sh.at[didx[mb]],
                                      sem_s.at[db]).wait()

            def scale(mb, db):
                data_b = data[db]

                def grp(g2, carry2):
                    rowv = lax.iota(i32, L) + g2 * L
                    cvec = coef_v[mb, pl.ds(g2 * L, L)]
                    for j in range(half):
                        jv = jnp.full((L,), j, i32)
                        v = plsc.load_gather(data_b, [rowv, jv])
                        plsc.store_scatter(data_b, [rowv, jv], v * cvec)
                    return carry2
                lax.fori_loop(0, 128 // L, grp, 0)

            meta_start(0, 0)
            meta_start(1, 1)
            meta_wait(0, 0)
            gather_start(0, 0)

            ngrp = spt // 4

            def group(g, carry):
                for u in range(4):
                    k = g * 4 + u
                    db, odb = u % 2, (u + 1) % 2
                    mb, mb1, mb2 = u, (u + 1) % 4, (u + 2) % 4
                    gather_wait(mb, db)
                    scale(mb, db)
                    scatter_start(mb, db)
                    if u > 0:
                        scatter_wait((u - 1) % 4, odb)
                    else:
                        @pl.when(g > 0)
                        def _():
                            scatter_wait(3, odb)
                    if u < 3:
                        meta_wait(k + 1, mb1)
                        gather_start(mb1, odb)
                    else:
                        @pl.when(g < ngrp - 1)
                        def _():
                            meta_wait(k + 1, mb1)
                            gather_start(mb1, odb)
                    if u < 2:
                        meta_start(k + 2, mb2)
                    else:
                        @pl.when(g < ngrp - 1)
                        def _():
                            meta_start(k + 2, mb2)
                return carry
            lax.fori_loop(0, ngrp, group, 0)

            scatter_wait((spt - 1) % 4, (spt - 1) % 2)

            plsc.subcore_barrier()
            pltpu.sync_copy(
                y_sh.at[pl.ds(sid * rows_t, rows_t)],
                yout_ref.at[pl.ds(cid * n_rows + sid * rows_t, rows_t)])

    ycat = _msg_kernel(xwflat, idxd, idxs, coef)
    ya = ycat[:n]
    yb = ycat[n_rows:n_rows + n]

    nb = n // bx
    out = pl.pallas_call(
        _fin_body,
        grid=(nb,),
        in_specs=[
            pl.BlockSpec((bx, half), lambda i: (i, 0)),
            pl.BlockSpec((bx, half), lambda i: (i, 0)),
            pl.BlockSpec((bx, half), lambda i: (i, 0)),
            pl.BlockSpec((bx, half), lambda i: (i, 0)),
            pl.BlockSpec((1, 1, bx), lambda i: (i, 0, 0)),
        ],
        out_specs=pl.BlockSpec((bx, d), lambda i: (i, 0)),
        out_shape=jax.ShapeDtypeStruct((n, d), f32),
    )(ya, yb, xw2[0], xw2[1], deg.reshape(nb, 1, bx))
    return out


# speedup vs baseline: 4.4098x; 4.4098x over previous
import functools

import jax
import jax.numpy as jnp
from jax import lax
from jax.experimental import pallas as pl
from jax.experimental.pallas import tpu as pltpu
from jax.experimental.pallas import tpu_sc as plsc

NC = 2
NS = 16
L = 16

_mesh = plsc.VectorSubcoreMesh(
    core_axis_name="c", subcore_axis_name="s", num_cores=NC, num_subcores=NS
)


def _softplus(r):
    return jnp.log(1.0 + jnp.exp(-jnp.abs(r))) + jnp.maximum(r, 0.0)


def _gate_body(ea_ref, gw1_ref, gb1_ref, gw2_ref, gb2_ref, o_ref):
    ea = ea_ref[...]
    h = lax.dot_general(ea, gw1_ref[...], (((1,), (1,)), ((), ())),
                        preferred_element_type=jnp.float32)
    h = jnp.maximum(h + gb1_ref[...], 0.0)
    s = jnp.sum(h * gw2_ref[...], axis=1) + gb2_ref[0, 0]
    o_ref[0, 0, :] = 1.0 / (1.0 + jnp.exp(-s))


def _xw_body(x_ref, wmu_ref, wrho_ref, epsw_ref, bmu_ref, brho_ref, epsb_ref,
             o_ref):
    w = wmu_ref[...] + _softplus(wrho_ref[...]) * epsw_ref[...]
    b = bmu_ref[...] + _softplus(brho_ref[...]) * epsb_ref[...]
    xw = lax.dot_general(x_ref[...], w, (((1,), (1,)), ((), ())),
                         preferred_element_type=jnp.float32) + b
    half = xw.shape[1] // 2
    o_ref[0] = xw[:, :half]
    o_ref[1] = xw[:, half:]


def _fin_body(ya_ref, yb_ref, xwa_ref, xwb_ref, deg_ref, o_ref):
    inv = 1.0 / deg_ref[0, 0, :]
    a = ya_ref[...] + xwa_ref[...] * inv[:, None]
    b = yb_ref[...] + xwb_ref[...] * inv[:, None]
    o_ref[...] = jnp.maximum(jnp.concatenate([a, b], axis=1), 0.0)


def _rsqrt_newton(p):
    i = plsc.bitcast(p, jnp.int32)
    i = jnp.int32(0x5F3759DF) - (i >> 1)
    y = plsc.bitcast(i, jnp.float32)
    for _ in range(3):
        y = y * (1.5 - 0.5 * p * y * y)
    return y


def kernel(x, edge_index, edge_attr, w_mu, w_rho, b_mu, b_rho, gW1, gb1, gW2,
           gb2, eps_w, eps_b):
    n, d = x.shape
    e = edge_index.shape[1]
    ef = edge_attr.shape[1]
    h_dim = gW1.shape[0]
    half = d // 2
    f32, i32 = jnp.float32, jnp.int32

    unit = NC * NS * 128
    e_pad = ((e + unit - 1) // unit) * unit
    ept = e_pad // (NC * NS)
    pad2 = 2 * e_pad
    spt = pad2 // 128 // NS
    c1 = 1024
    rows_t = ((n // NS + 7) // 8) * 8
    n_rows = NS * rows_t

    be = 20000
    g3 = pl.pallas_call(
        _gate_body,
        grid=(e // be,),
        in_specs=[
            pl.BlockSpec((be, ef), lambda i: (i, 0)),
            pl.BlockSpec((h_dim, ef), lambda i: (0, 0)),
            pl.BlockSpec((1, h_dim), lambda i: (0, 0)),
            pl.BlockSpec((1, h_dim), lambda i: (0, 0)),
            pl.BlockSpec((1, 1), lambda i: (0, 0)),
        ],
        out_specs=pl.BlockSpec((1, 1, be), lambda i: (i, 0, 0)),
        out_shape=jax.ShapeDtypeStruct((e // be, 1, be), f32),
    )(edge_attr, gW1, gb1.reshape(1, h_dim), gW2, gb2.reshape(1, 1))
    g = g3.reshape(e)

    bx = 1000
    xw2 = pl.pallas_call(
        _xw_body,
        grid=(n // bx,),
        in_specs=[
            pl.BlockSpec((bx, d), lambda i: (i, 0)),
            pl.BlockSpec((d, d), lambda i: (0, 0)),
            pl.BlockSpec((d, d), lambda i: (0, 0)),
            pl.BlockSpec((d, d), lambda i: (0, 0)),
            pl.BlockSpec((1, d), lambda i: (0, 0)),
            pl.BlockSpec((1, d), lambda i: (0, 0)),
            pl.BlockSpec((1, d), lambda i: (0, 0)),
        ],
        out_specs=pl.BlockSpec((2, bx, half), lambda i: (0, i, 0)),
        out_shape=jax.ShapeDtypeStruct((2, n, half), f32),
    )(x, w_mu, w_rho, eps_w, b_mu.reshape(1, d), b_rho.reshape(1, d),
      eps_b.reshape(1, d))
    xwflat = xw2.reshape(2 * n, half)

    zi = jnp.zeros((e_pad - e,), i32)
    zf = jnp.zeros((e_pad - e,), f32)
    ei0 = jnp.concatenate([edge_index[0].astype(i32), zi])
    ei1 = jnp.concatenate([edge_index[1].astype(i32), zi])
    gp = jnp.concatenate([g, zf])
    idxd = jnp.concatenate([ei0, ei1])
    idxs = jnp.concatenate([ei1, ei0])

    @functools.partial(
        pl.kernel,
        mesh=_mesh,
        compiler_params=pltpu.CompilerParams(needs_layout_passes=False),
        out_type=jax.ShapeDtypeStruct((NC, n), f32),
        scratch_types=[
            pltpu.VMEM((128,), i32),
            pltpu.VMEM((128,), f32),
            pltpu.VMEM((n,), f32),
            pltpu.VMEM_SHARED((n,), f32),
        ],
    )
    def _deg_kernel(ei0_ref, ei1_ref, g_ref, out_ref, idx_v, val_v, z_v, deg_sh):
        cid = lax.axis_index("c")
        sid = lax.axis_index("s")
        tid = sid * NC + cid

        @pl.when(sid == 0)
        def _():
            def zb(i, carry):
                z_v[pl.ds(i * L, L)] = jnp.zeros((L,), f32)
                return carry
            lax.fori_loop(0, n // L, zb, 0)
            pltpu.sync_copy(z_v, deg_sh)

        plsc.subcore_barrier()

        def chunk(k, carry):
            off = pl.multiple_of(tid * ept + k * 128, 128)
            pltpu.sync_copy(g_ref.at[pl.ds(off, 128)], val_v)
            pltpu.sync_copy(ei0_ref.at[pl.ds(off, 128)], idx_v)
            pltpu.sync_copy(val_v, deg_sh.at[idx_v], add=True)
            pltpu.sync_copy(ei1_ref.at[pl.ds(off, 128)], idx_v)
            pltpu.sync_copy(val_v, deg_sh.at[idx_v], add=True)
            return carry
        lax.fori_loop(0, ept // 128, chunk, 0)

        plsc.subcore_barrier()

        @pl.when(sid == 0)
        def _():
            pltpu.sync_copy(deg_sh, out_ref.at[cid])

    deg2 = _deg_kernel(ei0, ei1, gp)

    @functools.partial(
        pl.kernel,
        mesh=_mesh,
        compiler_params=pltpu.CompilerParams(needs_layout_passes=False),
        out_type=(
            jax.ShapeDtypeStruct((pad2,), f32),
            jax.ShapeDtypeStruct((n,), f32),
        ),
        scratch_types=[
            pltpu.VMEM((n,), f32),
            pltpu.VMEM((n,), f32),
            pltpu.VMEM((c1,), i32),
            pltpu.VMEM((c1,), i32),
            pltpu.VMEM((c1,), f32),
            pltpu.VMEM((c1,), f32),
        ],
    )
    def _coef_kernel(deg2_ref, ei0_ref, ei1_ref, g_ref, coef_ref, degout_ref,
                     deg_v, tmp_v, d_v, s_v, g_v, c_v):
        cid = lax.axis_index("c")
        sid = lax.axis_index("s")
        tid = sid * NC + cid
        pltpu.sync_copy(deg2_ref.at[0], deg_v)
        pltpu.sync_copy(deg2_ref.at[1], tmp_v)

        def addb(i, carry):
            sl = pl.ds(i * L, L)
            deg_v[sl] = deg_v[sl] + tmp_v[sl] + 1.0
            return carry
        lax.fori_loop(0, n // L, addb, 0)

        def chunk(j, carry):
            off = pl.multiple_of(tid * ept + j * c1, c1)
            pltpu.sync_copy(ei0_ref.at[pl.ds(off, c1)], d_v)
            pltpu.sync_copy(ei1_ref.at[pl.ds(off, c1)], s_v)
            pltpu.sync_copy(g_ref.at[pl.ds(off, c1)], g_v)

            def grp(i, carry2):
                sl = pl.ds(i * L, L)
                dd = plsc.load_gather(deg_v, [d_v[sl]])
                ds_ = plsc.load_gather(deg_v, [s_v[sl]])
                c_v[sl] = g_v[sl] * _rsqrt_newton(dd * ds_)
                return carry2
            lax.fori_loop(0, c1 // L, grp, 0)

            pltpu.sync_copy(c_v, coef_ref.at[pl.ds(off, c1)])
            off2 = pl.multiple_of(off + e_pad, c1)
            pltpu.sync_copy(c_v, coef_ref.at[pl.ds(off2, c1)])
            return carry
        lax.fori_loop(0, ept // c1, chunk, 0)

        @pl.when(jnp.logical_and(cid == 0, sid == 0))
        def _():
            pltpu.sync_copy(deg_v, degout_ref)

    coef, deg = _coef_kernel(deg2, ei0, ei1, gp)

    @functools.partial(
        pl.kernel,
        mesh=_mesh,
        compiler_params=pltpu.CompilerParams(needs_layout_passes=False),
        out_type=jax.ShapeDtypeStruct((2 * n_rows, half), f32),
        scratch_types=(
            [pltpu.VMEM((128, half), f32) for _ in range(2)]
            + [pltpu.VMEM((128,), i32) for _ in range(4)]
            + [
                pltpu.VMEM((4, 128), i32),
                pltpu.VMEM((4, 128), f32),
                pltpu.VMEM_SHARED((n_rows, half), f32),
                pltpu.SemaphoreType.DMA((4,)),
                pltpu.SemaphoreType.DMA((4,)),
                pltpu.SemaphoreType.DMA((2,)),
            ]
        ),
    )
    def _msg_kernel(tbl_ref, idxd_ref, idxs_ref, coef_ref,
                    yout_ref, data0, data1,
                    didx0, didx1, didx2, didx3, sidx_v, coef_v, y_sh,
                    sem_m, sem_g, sem_s):
        cid = lax.axis_index("c")
        sid = lax.axis_index("s")
        cbase = cid * n
        data = [data0, data1]
        didx = [didx0, didx1, didx2, didx3]

        def zrow(i, carry):
            for j in range(half // L):
                data0[i, pl.ds(j * L, L)] = jnp.zeros((L,), f32)
            return carry
        lax.fori_loop(0, 128, zrow, 0)

        nfull = rows_t // 128
        rem = rows_t - nfull * 128
        for q in range(nfull):
            pltpu.sync_copy(data0, y_sh.at[pl.ds(sid * rows_t + q * 128, 128)])
        if rem > 0:
            pltpu.sync_copy(data0.at[pl.ds(0, rem)],
                            y_sh.at[pl.ds(sid * rows_t + nfull * 128, rem)])

        plsc.subcore_barrier()

        if True:
            def _off(k):
                return pl.multiple_of((sid * spt + k) * 128, 128)

            def meta_start(k, mb):
                off = _off(k)
                pltpu.async_copy(idxs_ref.at[pl.ds(off, 128)],
                                 sidx_v.at[mb], sem_m.at[mb])
                pltpu.async_copy(idxd_ref.at[pl.ds(off, 128)],
                                 didx[mb], sem_m.at[mb])
                pltpu.async_copy(coef_ref.at[pl.ds(off, 128)],
                                 coef_v.at[mb], sem_m.at[mb])

            def meta_wait(k, mb):
                off = _off(k)
                pltpu.make_async_copy(idxs_ref.at[pl.ds(off, 128)],
                                      sidx_v.at[mb], sem_m.at[mb]).wait()
                pltpu.make_async_copy(idxd_ref.at[pl.ds(off, 128)],
                                      didx[mb], sem_m.at[mb]).wait()
                pltpu.make_async_copy(coef_ref.at[pl.ds(off, 128)],
                                      coef_v.at[mb], sem_m.at[mb]).wait()
                for j2 in range(128 // L):
                    sl = pl.ds(j2 * L, L)
                    sidx_v[mb, sl] = sidx_v[mb, sl] + cbase

            def gather_start(mb, db):
                pltpu.async_copy(tbl_ref.at[sidx_v.at[mb]], data[db],
                                 sem_g.at[mb])

            def gather_wait(mb, db):
                pltpu.make_async_copy(tbl_ref.at[sidx_v.at[mb]], data[db],
                                      sem_g.at[mb]).wait()

            def scatter_start(mb, db):
                pltpu.async_copy(data[db], y_sh.at[didx[mb]], sem_s.at[db],
                                 add=True)

            def scatter_wait(mb, db):
                pltpu.make_async_copy(data[db], y_sh.at[didx[mb]],
                                      sem_s.at[db]).wait()

            def scale(mb, db):
                data_b = data[db]

                def grp(g2, carry2):
                    cvec16 = coef_v[mb, pl.ds(g2 * L, L)]
                    for t in range(L):
                        tb = g2 * L + t
                        bc = cvec16.at[jnp.full((L,), t, i32)].get(
                            mode="promise_in_bounds")
                        for j in range(half // L):
                            sl = pl.ds(j * L, L)
                            data_b[tb, sl] = data_b[tb, sl] * bc
                    return carry2
                lax.fori_loop(0, 128 // L, grp, 0)

            meta_start(0, 0)
            meta_start(1, 1)
            meta_wait(0, 0)
            gather_start(0, 0)

            ngrp = spt // 4

            def group(g, carry):
                for u in range(4):
                    k = g * 4 + u
                    db, odb = u % 2, (u + 1) % 2
                    mb, mb1, mb2 = u, (u + 1) % 4, (u + 2) % 4
                    gather_wait(mb, db)
                    scale(mb, db)
                    scatter_start(mb, db)
                    if u > 0:
                        scatter_wait((u - 1) % 4, odb)
                    else:
                        @pl.when(g > 0)
                        def _():
                            scatter_wait(3, odb)
                    if u < 3:
                        meta_wait(k + 1, mb1)
                        gather_start(mb1, odb)
                    else:
                        @pl.when(g < ngrp - 1)
                        def _():
                            meta_wait(k + 1, mb1)
                            gather_start(mb1, odb)
                    if u < 2:
                        meta_start(k + 2, mb2)
                    else:
                        @pl.when(g < ngrp - 1)
                        def _():
                            meta_start(k + 2, mb2)
                return carry
            lax.fori_loop(0, ngrp, group, 0)

            scatter_wait((spt - 1) % 4, (spt - 1) % 2)

            plsc.subcore_barrier()
            pltpu.sync_copy(
                y_sh.at[pl.ds(sid * rows_t, rows_t)],
                yout_ref.at[pl.ds(cid * n_rows + sid * rows_t, rows_t)])

    ycat = _msg_kernel(xwflat, idxd, idxs, coef)
    ya = ycat[:n]
    yb = ycat[n_rows:n_rows + n]

    nb = n // bx
    out = pl.pallas_call(
        _fin_body,
        grid=(nb,),
        in_specs=[
            pl.BlockSpec((bx, half), lambda i: (i, 0)),
            pl.BlockSpec((bx, half), lambda i: (i, 0)),
            pl.BlockSpec((bx, half), lambda i: (i, 0)),
            pl.BlockSpec((bx, half), lambda i: (i, 0)),
            pl.BlockSpec((1, 1, bx), lambda i: (i, 0, 0)),
        ],
        out_specs=pl.BlockSpec((bx, d), lambda i: (i, 0)),
        out_shape=jax.ShapeDtypeStruct((n, d), f32),
    )(ya, yb, xw2[0], xw2[1], deg.reshape(nb, 1, bx))
    return out


# speedup vs baseline: 4.8038x; 1.0893x over previous
import functools

import jax
import jax.numpy as jnp
from jax import lax
from jax.experimental import pallas as pl
from jax.experimental.pallas import tpu as pltpu
from jax.experimental.pallas import tpu_sc as plsc

NC = 2
NS = 16
L = 16

_mesh = plsc.VectorSubcoreMesh(
    core_axis_name="c", subcore_axis_name="s", num_cores=NC, num_subcores=NS
)


def _softplus(r):
    return jnp.log(1.0 + jnp.exp(-jnp.abs(r))) + jnp.maximum(r, 0.0)


def _gate_body(ea_ref, gw1_ref, gb1_ref, gw2_ref, gb2_ref, o_ref):
    ea = ea_ref[...]
    h = lax.dot_general(ea, gw1_ref[...], (((1,), (1,)), ((), ())),
                        preferred_element_type=jnp.float32)
    h = jnp.maximum(h + gb1_ref[...], 0.0)
    s = jnp.sum(h * gw2_ref[...], axis=1) + gb2_ref[0, 0]
    o_ref[0, 0, :] = 1.0 / (1.0 + jnp.exp(-s))


def _xw_body(x_ref, wmu_ref, wrho_ref, epsw_ref, bmu_ref, brho_ref, epsb_ref,
             o_ref):
    w = wmu_ref[...] + _softplus(wrho_ref[...]) * epsw_ref[...]
    b = bmu_ref[...] + _softplus(brho_ref[...]) * epsb_ref[...]
    xw = lax.dot_general(x_ref[...], w, (((1,), (1,)), ((), ())),
                         preferred_element_type=jnp.float32) + b
    half = xw.shape[1] // 2
    o_ref[0] = xw[:, :half]
    o_ref[1] = xw[:, half:]


def _fin_body(ya_ref, yb_ref, xwa_ref, xwb_ref, deg_ref, o_ref):
    inv = 1.0 / deg_ref[0, 0, :]
    a = ya_ref[...] + xwa_ref[...] * inv[:, None]
    b = yb_ref[...] + xwb_ref[...] * inv[:, None]
    o_ref[...] = jnp.maximum(jnp.concatenate([a, b], axis=1), 0.0)


def _rsqrt_newton(p):
    i = plsc.bitcast(p, jnp.int32)
    i = jnp.int32(0x5F3759DF) - (i >> 1)
    y = plsc.bitcast(i, jnp.float32)
    for _ in range(3):
        y = y * (1.5 - 0.5 * p * y * y)
    return y


def kernel(x, edge_index, edge_attr, w_mu, w_rho, b_mu, b_rho, gW1, gb1, gW2,
           gb2, eps_w, eps_b):
    n, d = x.shape
    e = edge_index.shape[1]
    ef = edge_attr.shape[1]
    h_dim = gW1.shape[0]
    half = d // 2
    f32, i32 = jnp.float32, jnp.int32

    unit = NC * NS * 128
    e_pad = ((e + unit - 1) // unit) * unit
    ept = e_pad // (NC * NS)
    pad2 = 2 * e_pad
    spt = pad2 // 128 // NS
    c1 = 1024
    rows_t = ((n // NS + 7) // 8) * 8
    n_rows = NS * rows_t

    be = 20000
    g3 = pl.pallas_call(
        _gate_body,
        grid=(e // be,),
        in_specs=[
            pl.BlockSpec((be, ef), lambda i: (i, 0)),
            pl.BlockSpec((h_dim, ef), lambda i: (0, 0)),
            pl.BlockSpec((1, h_dim), lambda i: (0, 0)),
            pl.BlockSpec((1, h_dim), lambda i: (0, 0)),
            pl.BlockSpec((1, 1), lambda i: (0, 0)),
        ],
        out_specs=pl.BlockSpec((1, 1, be), lambda i: (i, 0, 0)),
        out_shape=jax.ShapeDtypeStruct((e // be, 1, be), f32),
    )(edge_attr, gW1, gb1.reshape(1, h_dim), gW2, gb2.reshape(1, 1))
    g = g3.reshape(e)

    bx = 1000
    xw2 = pl.pallas_call(
        _xw_body,
        grid=(n // bx,),
        in_specs=[
            pl.BlockSpec((bx, d), lambda i: (i, 0)),
            pl.BlockSpec((d, d), lambda i: (0, 0)),
            pl.BlockSpec((d, d), lambda i: (0, 0)),
            pl.BlockSpec((d, d), lambda i: (0, 0)),
            pl.BlockSpec((1, d), lambda i: (0, 0)),
            pl.BlockSpec((1, d), lambda i: (0, 0)),
            pl.BlockSpec((1, d), lambda i: (0, 0)),
        ],
        out_specs=pl.BlockSpec((2, bx, half), lambda i: (0, i, 0)),
        out_shape=jax.ShapeDtypeStruct((2, n, half), f32),
    )(x, w_mu, w_rho, eps_w, b_mu.reshape(1, d), b_rho.reshape(1, d),
      eps_b.reshape(1, d))
    xwflat = xw2.reshape(2 * n, half)

    zi = jnp.zeros((e_pad - e,), i32)
    zf = jnp.zeros((e_pad - e,), f32)
    ei0 = jnp.concatenate([edge_index[0].astype(i32), zi])
    ei1 = jnp.concatenate([edge_index[1].astype(i32), zi])
    gp = jnp.concatenate([g, zf])
    idxd = jnp.concatenate([ei0, ei1])
    idxs = jnp.concatenate([ei1, ei0])

    @functools.partial(
        pl.kernel,
        mesh=_mesh,
        compiler_params=pltpu.CompilerParams(needs_layout_passes=False),
        out_type=jax.ShapeDtypeStruct((NC, n), f32),
        scratch_types=[
            pltpu.VMEM((128,), i32),
            pltpu.VMEM((128,), f32),
            pltpu.VMEM((n,), f32),
            pltpu.VMEM_SHARED((n,), f32),
        ],
    )
    def _deg_kernel(ei0_ref, ei1_ref, g_ref, out_ref, idx_v, val_v, z_v, deg_sh):
        cid = lax.axis_index("c")
        sid = lax.axis_index("s")
        tid = sid * NC + cid

        @pl.when(sid == 0)
        def _():
            def zb(i, carry):
                z_v[pl.ds(i * L, L)] = jnp.zeros((L,), f32)
                return carry
            lax.fori_loop(0, n // L, zb, 0)
            pltpu.sync_copy(z_v, deg_sh)

        plsc.subcore_barrier()

        def chunk(k, carry):
            off = pl.multiple_of(tid * ept + k * 128, 128)
            pltpu.sync_copy(g_ref.at[pl.ds(off, 128)], val_v)
            pltpu.sync_copy(ei0_ref.at[pl.ds(off, 128)], idx_v)
            pltpu.sync_copy(val_v, deg_sh.at[idx_v], add=True)
            pltpu.sync_copy(ei1_ref.at[pl.ds(off, 128)], idx_v)
            pltpu.sync_copy(val_v, deg_sh.at[idx_v], add=True)
            return carry
        lax.fori_loop(0, ept // 128, chunk, 0)

        plsc.subcore_barrier()

        @pl.when(sid == 0)
        def _():
            pltpu.sync_copy(deg_sh, out_ref.at[cid])

    deg2 = _deg_kernel(ei0, ei1, gp)

    @functools.partial(
        pl.kernel,
        mesh=_mesh,
        compiler_params=pltpu.CompilerParams(needs_layout_passes=False),
        out_type=(
            jax.ShapeDtypeStruct((pad2,), f32),
            jax.ShapeDtypeStruct((n,), f32),
        ),
        scratch_types=[
            pltpu.VMEM((n,), f32),
            pltpu.VMEM((n,), f32),
            pltpu.VMEM((c1,), i32),
            pltpu.VMEM((c1,), i32),
            pltpu.VMEM((c1,), f32),
            pltpu.VMEM((c1,), f32),
        ],
    )
    def _coef_kernel(deg2_ref, ei0_ref, ei1_ref, g_ref, coef_ref, degout_ref,
                     deg_v, tmp_v, d_v, s_v, g_v, c_v):
        cid = lax.axis_index("c")
        sid = lax.axis_index("s")
        tid = sid * NC + cid
        pltpu.sync_copy(deg2_ref.at[0], deg_v)
        pltpu.sync_copy(deg2_ref.at[1], tmp_v)

        def addb(i, carry):
            sl = pl.ds(i * L, L)
            deg_v[sl] = deg_v[sl] + tmp_v[sl] + 1.0
            return carry
        lax.fori_loop(0, n // L, addb, 0)

        def chunk(j, carry):
            off = pl.multiple_of(tid * ept + j * c1, c1)
            pltpu.sync_copy(ei0_ref.at[pl.ds(off, c1)], d_v)
            pltpu.sync_copy(ei1_ref.at[pl.ds(off, c1)], s_v)
            pltpu.sync_copy(g_ref.at[pl.ds(off, c1)], g_v)

            def grp(i, carry2):
                sl = pl.ds(i * L, L)
                dd = plsc.load_gather(deg_v, [d_v[sl]])
                ds_ = plsc.load_gather(deg_v, [s_v[sl]])
                c_v[sl] = g_v[sl] * _rsqrt_newton(dd * ds_)
                return carry2
            lax.fori_loop(0, c1 // L, grp, 0)

            pltpu.sync_copy(c_v, coef_ref.at[pl.ds(off, c1)])
            off2 = pl.multiple_of(off + e_pad, c1)
            pltpu.sync_copy(c_v, coef_ref.at[pl.ds(off2, c1)])
            return carry
        lax.fori_loop(0, ept // c1, chunk, 0)

        @pl.when(jnp.logical_and(cid == 0, sid == 0))
        def _():
            pltpu.sync_copy(deg_v, degout_ref)

    coef, deg = _coef_kernel(deg2, ei0, ei1, gp)

    @functools.partial(
        pl.kernel,
        mesh=_mesh,
        compiler_params=pltpu.CompilerParams(needs_layout_passes=False),
        out_type=jax.ShapeDtypeStruct((2 * n_rows, half), f32),
        scratch_types=(
            [pltpu.VMEM((128, half), f32) for _ in range(2)]
            + [pltpu.VMEM((128,), i32) for _ in range(4)]
            + [
                pltpu.VMEM((4, 128), i32),
                pltpu.VMEM((4, 128), f32),
                pltpu.VMEM_SHARED((n_rows, half), f32),
                pltpu.SemaphoreType.DMA((4,)),
                pltpu.SemaphoreType.DMA((4,)),
                pltpu.SemaphoreType.DMA((2,)),
            ]
        ),
    )
    def _msg_kernel(tbl_ref, idxd_ref, idxs_ref, coef_ref,
                    yout_ref, data0, data1,
                    didx0, didx1, didx2, didx3, sidx_v, coef_v, y_sh,
                    sem_m, sem_g, sem_s):
        cid = lax.axis_index("c")
        sid = lax.axis_index("s")
        cbase = cid * n
        data = [data0, data1]
        didx = [didx0, didx1, didx2, didx3]

        def zrow(i, carry):
            for j in range(half // L):
                data0[i, pl.ds(j * L, L)] = jnp.zeros((L,), f32)
            return carry
        lax.fori_loop(0, 128, zrow, 0)

        nfull = rows_t // 128
        rem = rows_t - nfull * 128
        for q in range(nfull):
            pltpu.sync_copy(data0, y_sh.at[pl.ds(sid * rows_t + q * 128, 128)])
        if rem > 0:
            pltpu.sync_copy(data0.at[pl.ds(0, rem)],
                            y_sh.at[pl.ds(sid * rows_t + nfull * 128, rem)])

        plsc.subcore_barrier()

        if True:
            def _off(k):
                return pl.multiple_of((sid * spt + k) * 128, 128)

            def meta_start(k, mb):
                off = _off(k)
                pltpu.async_copy(idxs_ref.at[pl.ds(off, 128)],
                                 sidx_v.at[mb], sem_m.at[mb])
                pltpu.async_copy(idxd_ref.at[pl.ds(off, 128)],
                                 didx[mb], sem_m.at[mb])
                pltpu.async_copy(coef_ref.at[pl.ds(off, 128)],
                                 coef_v.at[mb], sem_m.at[mb])

            def meta_wait(k, mb):
                off = _off(k)
                pltpu.make_async_copy(idxs_ref.at[pl.ds(off, 128)],
                                      sidx_v.at[mb], sem_m.at[mb]).wait()
                pltpu.make_async_copy(idxd_ref.at[pl.ds(off, 128)],
                                      didx[mb], sem_m.at[mb]).wait()
                pltpu.make_async_copy(coef_ref.at[pl.ds(off, 128)],
                                      coef_v.at[mb], sem_m.at[mb]).wait()
                for j2 in range(128 // L):
                    sl = pl.ds(j2 * L, L)
                    sidx_v[mb, sl] = sidx_v[mb, sl] + cbase

            def gather_start(mb, db):
                pltpu.async_copy(tbl_ref.at[sidx_v.at[mb]], data[db],
                                 sem_g.at[mb])

            def gather_wait(mb, db):
                pltpu.make_async_copy(tbl_ref.at[sidx_v.at[mb]], data[db],
                                      sem_g.at[mb]).wait()

            def scatter_start(mb, db):
                pltpu.async_copy(data[db], y_sh.at[didx[mb]], sem_s.at[db],
                                 add=True)

            def scatter_wait(mb, db):
                pltpu.make_async_copy(data[db], y_sh.at[didx[mb]],
                                      sem_s.at[db]).wait()

            def scale(mb, db):
                data_b = data[db]

                def grp(g2, carry2):
                    cvec16 = coef_v[mb, pl.ds(g2 * L, L)]
                    for t in range(L):
                        tb = g2 * L + t
                        bc = cvec16.at[jnp.full((L,), t, i32)].get(
                            mode="promise_in_bounds")
                        for j in range(half // L):
                            sl = pl.ds(j * L, L)
                            data_b[tb, sl] = data_b[tb, sl] * bc
                    return carry2
                lax.fori_loop(0, 128 // L, grp, 0)

            meta_start(0, 0)
            meta_start(1, 1)
            meta_wait(0, 0)
            gather_start(0, 0)

            ngrp = spt // 4

            def group(g, carry):
                for u in range(4):
                    k = g * 4 + u
                    db, odb = u % 2, (u + 1) % 2
                    mb, mb1, mb2 = u, (u + 1) % 4, (u + 2) % 4
                    gather_wait(mb, db)
                    if u > 0:
                        scatter_wait((u - 1) % 4, odb)
                    else:
                        @pl.when(g > 0)
                        def _():
                            scatter_wait(3, odb)
                    if u < 3:
                        meta_wait(k + 1, mb1)
                        gather_start(mb1, odb)
                    else:
                        @pl.when(g < ngrp - 1)
                        def _():
                            meta_wait(k + 1, mb1)
                            gather_start(mb1, odb)
                    scale(mb, db)
                    scatter_start(mb, db)
                    if u < 2:
                        meta_start(k + 2, mb2)
                    else:
                        @pl.when(g < ngrp - 1)
                        def _():
                            meta_start(k + 2, mb2)
                return carry
            lax.fori_loop(0, ngrp, group, 0)

            scatter_wait((spt - 1) % 4, (spt - 1) % 2)

            plsc.subcore_barrier()
            pltpu.sync_copy(
                y_sh.at[pl.ds(sid * rows_t, rows_t)],
                yout_ref.at[pl.ds(cid * n_rows + sid * rows_t, rows_t)])

    ycat = _msg_kernel(xwflat, idxd, idxs, coef)
    ya = ycat[:n]
    yb = ycat[n_rows:n_rows + n]

    nb = n // bx
    out = pl.pallas_call(
        _fin_body,
        grid=(nb,),
        in_specs=[
            pl.BlockSpec((bx, half), lambda i: (i, 0)),
            pl.BlockSpec((bx, half), lambda i: (i, 0)),
            pl.BlockSpec((bx, half), lambda i: (i, 0)),
            pl.BlockSpec((bx, half), lambda i: (i, 0)),
            pl.BlockSpec((1, 1, bx), lambda i: (i, 0, 0)),
        ],
        out_specs=pl.BlockSpec((bx, d), lambda i: (i, 0)),
        out_shape=jax.ShapeDtypeStruct((n, d), f32),
    )(ya, yb, xw2[0], xw2[1], deg.reshape(nb, 1, bx))
    return out


# speedup vs baseline: 5.0684x; 1.0551x over previous
import functools

import jax
import jax.numpy as jnp
from jax import lax
from jax.experimental import pallas as pl
from jax.experimental.pallas import tpu as pltpu
from jax.experimental.pallas import tpu_sc as plsc

NC = 2
NS = 16
L = 16

_mesh = plsc.VectorSubcoreMesh(
    core_axis_name="c", subcore_axis_name="s", num_cores=NC, num_subcores=NS
)


def _softplus(r):
    return jnp.log(1.0 + jnp.exp(-jnp.abs(r))) + jnp.maximum(r, 0.0)


def _gate_body(ea_ref, gw1_ref, gb1_ref, gw2_ref, gb2_ref, o_ref):
    ea = ea_ref[...]
    h = lax.dot_general(ea, gw1_ref[...], (((1,), (1,)), ((), ())),
                        preferred_element_type=jnp.float32)
    h = jnp.maximum(h + gb1_ref[...], 0.0)
    s = jnp.sum(h * gw2_ref[...], axis=1) + gb2_ref[0, 0]
    o_ref[0, 0, :] = 1.0 / (1.0 + jnp.exp(-s))


def _xw_body(x_ref, wmu_ref, wrho_ref, epsw_ref, bmu_ref, brho_ref, epsb_ref,
             o_ref):
    w = wmu_ref[...] + _softplus(wrho_ref[...]) * epsw_ref[...]
    b = bmu_ref[...] + _softplus(brho_ref[...]) * epsb_ref[...]
    xw = lax.dot_general(x_ref[...], w, (((1,), (1,)), ((), ())),
                         preferred_element_type=jnp.float32) + b
    half = xw.shape[1] // 2
    o_ref[0] = xw[:, :half]
    o_ref[1] = xw[:, half:]


def _fin_body(ya_ref, yb_ref, xwa_ref, xwb_ref, deg_ref, o_ref):
    inv = 1.0 / deg_ref[0, 0, :]
    a = ya_ref[...] + xwa_ref[...] * inv[:, None]
    b = yb_ref[...] + xwb_ref[...] * inv[:, None]
    o_ref[...] = jnp.maximum(jnp.concatenate([a, b], axis=1), 0.0)


def _rsqrt_newton(p):
    i = plsc.bitcast(p, jnp.int32)
    i = jnp.int32(0x5F3759DF) - (i >> 1)
    y = plsc.bitcast(i, jnp.float32)
    for _ in range(3):
        y = y * (1.5 - 0.5 * p * y * y)
    return y


def kernel(x, edge_index, edge_attr, w_mu, w_rho, b_mu, b_rho, gW1, gb1, gW2,
           gb2, eps_w, eps_b):
    n, d = x.shape
    e = edge_index.shape[1]
    ef = edge_attr.shape[1]
    h_dim = gW1.shape[0]
    half = d // 2
    f32, i32 = jnp.float32, jnp.int32

    unit = NC * NS * 128
    e_pad = ((e + unit - 1) // unit) * unit
    ept = e_pad // (NC * NS)
    pad2 = 2 * e_pad
    spt = pad2 // 128 // NS
    c1 = 1024
    rows_t = ((n // NS + 7) // 8) * 8
    n_rows = NS * rows_t

    be = 20000
    g3 = pl.pallas_call(
        _gate_body,
        grid=(e // be,),
        in_specs=[
            pl.BlockSpec((be, ef), lambda i: (i, 0)),
            pl.BlockSpec((h_dim, ef), lambda i: (0, 0)),
            pl.BlockSpec((1, h_dim), lambda i: (0, 0)),
            pl.BlockSpec((1, h_dim), lambda i: (0, 0)),
            pl.BlockSpec((1, 1), lambda i: (0, 0)),
        ],
        out_specs=pl.BlockSpec((1, 1, be), lambda i: (i, 0, 0)),
        out_shape=jax.ShapeDtypeStruct((e // be, 1, be), f32),
    )(edge_attr, gW1, gb1.reshape(1, h_dim), gW2, gb2.reshape(1, 1))
    g = g3.reshape(e)

    bx = 1000
    xw2 = pl.pallas_call(
        _xw_body,
        grid=(n // bx,),
        in_specs=[
            pl.BlockSpec((bx, d), lambda i: (i, 0)),
            pl.BlockSpec((d, d), lambda i: (0, 0)),
            pl.BlockSpec((d, d), lambda i: (0, 0)),
            pl.BlockSpec((d, d), lambda i: (0, 0)),
            pl.BlockSpec((1, d), lambda i: (0, 0)),
            pl.BlockSpec((1, d), lambda i: (0, 0)),
            pl.BlockSpec((1, d), lambda i: (0, 0)),
        ],
        out_specs=pl.BlockSpec((2, bx, half), lambda i: (0, i, 0)),
        out_shape=jax.ShapeDtypeStruct((2, n, half), f32),
    )(x, w_mu, w_rho, eps_w, b_mu.reshape(1, d), b_rho.reshape(1, d),
      eps_b.reshape(1, d))
    xwflat = xw2.reshape(2 * n, half)

    zi = jnp.zeros((e_pad - e,), i32)
    zf = jnp.zeros((e_pad - e,), f32)
    ei0 = jnp.concatenate([edge_index[0].astype(i32), zi])
    ei1 = jnp.concatenate([edge_index[1].astype(i32), zi])
    gp = jnp.concatenate([g, zf])
    idxd = jnp.concatenate([ei0, ei1])
    idxs = jnp.concatenate([ei1, ei0])

    @functools.partial(
        pl.kernel,
        mesh=_mesh,
        compiler_params=pltpu.CompilerParams(needs_layout_passes=False),
        out_type=jax.ShapeDtypeStruct((NC, n), f32),
        scratch_types=[
            pltpu.VMEM((128,), i32),
            pltpu.VMEM((128,), i32),
            pltpu.VMEM((128,), i32),
            pltpu.VMEM((128,), i32),
            pltpu.VMEM((2, 128), f32),
            pltpu.VMEM((n,), f32),
            pltpu.VMEM_SHARED((n,), f32),
            pltpu.SemaphoreType.DMA((2,)),
            pltpu.SemaphoreType.DMA((2,)),
        ],
    )
    def _deg_kernel(ei0_ref, ei1_ref, g_ref, out_ref, i0a, i0b, i1a, i1b,
                    val_v, z_v, deg_sh, sem_m, sem_s):
        cid = lax.axis_index("c")
        sid = lax.axis_index("s")
        tid = sid * NC + cid
        i0 = [i0a, i0b]
        i1 = [i1a, i1b]

        @pl.when(sid == 0)
        def _():
            def zb(i, carry):
                z_v[pl.ds(i * L, L)] = jnp.zeros((L,), f32)
                return carry
            lax.fori_loop(0, n // L, zb, 0)
            pltpu.sync_copy(z_v, deg_sh)

        plsc.subcore_barrier()

        def _off(k):
            return pl.multiple_of(tid * ept + k * 128, 128)

        def meta_start(k, b):
            off = _off(k)
            pltpu.async_copy(g_ref.at[pl.ds(off, 128)], val_v.at[b],
                             sem_m.at[b])
            pltpu.async_copy(ei0_ref.at[pl.ds(off, 128)], i0[b], sem_m.at[b])
            pltpu.async_copy(ei1_ref.at[pl.ds(off, 128)], i1[b], sem_m.at[b])

        def meta_wait(k, b):
            off = _off(k)
            pltpu.make_async_copy(g_ref.at[pl.ds(off, 128)], val_v.at[b],
                                  sem_m.at[b]).wait()
            pltpu.make_async_copy(ei0_ref.at[pl.ds(off, 128)], i0[b],
                                  sem_m.at[b]).wait()
            pltpu.make_async_copy(ei1_ref.at[pl.ds(off, 128)], i1[b],
                                  sem_m.at[b]).wait()

        def sc_start(b):
            pltpu.async_copy(val_v.at[b], deg_sh.at[i0[b]], sem_s.at[b],
                             add=True)
            pltpu.async_copy(val_v.at[b], deg_sh.at[i1[b]], sem_s.at[b],
                             add=True)

        def sc_wait(b):
            pltpu.make_async_copy(val_v.at[b], deg_sh.at[i0[b]],
                                  sem_s.at[b]).wait()
            pltpu.make_async_copy(val_v.at[b], deg_sh.at[i1[b]],
                                  sem_s.at[b]).wait()

        nchunk = ept // 128
        meta_start(0, 0)

        def pair(p, carry):
            for u in range(2):
                k = p * 2 + u
                b, ob = u, 1 - u
                meta_wait(k, b)
                if u > 0:
                    sc_wait(ob)
                else:
                    @pl.when(p > 0)
                    def _():
                        sc_wait(ob)
                if u == 0:
                    meta_start(k + 1, ob)
                else:
                    @pl.when(p < nchunk // 2 - 1)
                    def _():
                        meta_start(k + 1, ob)
                sc_start(b)
            return carry
        lax.fori_loop(0, nchunk // 2, pair, 0)
        sc_wait(1)

        plsc.subcore_barrier()

        @pl.when(sid == 0)
        def _():
            pltpu.sync_copy(deg_sh, out_ref.at[cid])

    deg2 = _deg_kernel(ei0, ei1, gp)

    @functools.partial(
        pl.kernel,
        mesh=_mesh,
        compiler_params=pltpu.CompilerParams(needs_layout_passes=False),
        out_type=(
            jax.ShapeDtypeStruct((pad2,), f32),
            jax.ShapeDtypeStruct((n,), f32),
        ),
        scratch_types=[
            pltpu.VMEM((n,), f32),
            pltpu.VMEM((n,), f32),
            pltpu.VMEM((c1,), i32),
            pltpu.VMEM((c1,), i32),
            pltpu.VMEM((c1,), f32),
            pltpu.VMEM((c1,), f32),
        ],
    )
    def _coef_kernel(deg2_ref, ei0_ref, ei1_ref, g_ref, coef_ref, degout_ref,
                     deg_v, tmp_v, d_v, s_v, g_v, c_v):
        cid = lax.axis_index("c")
        sid = lax.axis_index("s")
        tid = sid * NC + cid
        pltpu.sync_copy(deg2_ref.at[0], deg_v)
        pltpu.sync_copy(deg2_ref.at[1], tmp_v)

        def addb(i, carry):
            sl = pl.ds(i * L, L)
            deg_v[sl] = deg_v[sl] + tmp_v[sl] + 1.0
            return carry
        lax.fori_loop(0, n // L, addb, 0)

        def chunk(j, carry):
            off = pl.multiple_of(tid * ept + j * c1, c1)
            pltpu.sync_copy(ei0_ref.at[pl.ds(off, c1)], d_v)
            pltpu.sync_copy(ei1_ref.at[pl.ds(off, c1)], s_v)
            pltpu.sync_copy(g_ref.at[pl.ds(off, c1)], g_v)

            def grp(i, carry2):
                sl = pl.ds(i * L, L)
                dd = plsc.load_gather(deg_v, [d_v[sl]])
                ds_ = plsc.load_gather(deg_v, [s_v[sl]])
                c_v[sl] = g_v[sl] * _rsqrt_newton(dd * ds_)
                return carry2
            lax.fori_loop(0, c1 // L, grp, 0)

            pltpu.sync_copy(c_v, coef_ref.at[pl.ds(off, c1)])
            off2 = pl.multiple_of(off + e_pad, c1)
            pltpu.sync_copy(c_v, coef_ref.at[pl.ds(off2, c1)])
            return carry
        lax.fori_loop(0, ept // c1, chunk, 0)

        @pl.when(jnp.logical_and(cid == 0, sid == 0))
        def _():
            pltpu.sync_copy(deg_v, degout_ref)

    coef, deg = _coef_kernel(deg2, ei0, ei1, gp)

    @functools.partial(
        pl.kernel,
        mesh=_mesh,
        compiler_params=pltpu.CompilerParams(needs_layout_passes=False),
        out_type=jax.ShapeDtypeStruct((2 * n_rows, half), f32),
        scratch_types=(
            [pltpu.VMEM((128, half), f32) for _ in range(2)]
            + [pltpu.VMEM((128,), i32) for _ in range(4)]
            + [
                pltpu.VMEM((4, 128), i32),
                pltpu.VMEM((4, 128), f32),
                pltpu.VMEM_SHARED((n_rows, half), f32),
                pltpu.SemaphoreType.DMA((4,)),
                pltpu.SemaphoreType.DMA((4,)),
                pltpu.SemaphoreType.DMA((2,)),
            ]
        ),
    )
    def _msg_kernel(tbl_ref, idxd_ref, idxs_ref, coef_ref,
                    yout_ref, data0, data1,
                    didx0, didx1, didx2, didx3, sidx_v, coef_v, y_sh,
                    sem_m, sem_g, sem_s):
        cid = lax.axis_index("c")
        sid = lax.axis_index("s")
        cbase = cid * n
        data = [data0, data1]
        didx = [didx0, didx1, didx2, didx3]

        def zrow(i, carry):
            for j in range(half // L):
                data0[i, pl.ds(j * L, L)] = jnp.zeros((L,), f32)
            return carry
        lax.fori_loop(0, 128, zrow, 0)

        nfull = rows_t // 128
        rem = rows_t - nfull * 128
        for q in range(nfull):
            pltpu.sync_copy(data0, y_sh.at[pl.ds(sid * rows_t + q * 128, 128)])
        if rem > 0:
            pltpu.sync_copy(data0.at[pl.ds(0, rem)],
                            y_sh.at[pl.ds(sid * rows_t + nfull * 128, rem)])

        plsc.subcore_barrier()

        if True:
            def _off(k):
                return pl.multiple_of((sid * spt + k) * 128, 128)

            def meta_start(k, mb):
                off = _off(k)
                pltpu.async_copy(idxs_ref.at[pl.ds(off, 128)],
                                 sidx_v.at[mb], sem_m.at[mb])
                pltpu.async_copy(idxd_ref.at[pl.ds(off, 128)],
                                 didx[mb], sem_m.at[mb])
                pltpu.async_copy(coef_ref.at[pl.ds(off, 128)],
                                 coef_v.at[mb], sem_m.at[mb])

            def meta_wait(k, mb):
                off = _off(k)
                pltpu.make_async_copy(idxs_ref.at[pl.ds(off, 128)],
                                      sidx_v.at[mb], sem_m.at[mb]).wait()
                pltpu.make_async_copy(idxd_ref.at[pl.ds(off, 128)],
                                      didx[mb], sem_m.at[mb]).wait()
                pltpu.make_async_copy(coef_ref.at[pl.ds(off, 128)],
                                      coef_v.at[mb], sem_m.at[mb]).wait()
                for j2 in range(128 // L):
                    sl = pl.ds(j2 * L, L)
                    sidx_v[mb, sl] = sidx_v[mb, sl] + cbase

            def gather_start(mb, db):
                pltpu.async_copy(tbl_ref.at[sidx_v.at[mb]], data[db],
                                 sem_g.at[mb])

            def gather_wait(mb, db):
                pltpu.make_async_copy(tbl_ref.at[sidx_v.at[mb]], data[db],
                                      sem_g.at[mb]).wait()

            def scatter_start(mb, db):
                pltpu.async_copy(data[db], y_sh.at[didx[mb]], sem_s.at[db],
                                 add=True)

            def scatter_wait(mb, db):
                pltpu.make_async_copy(data[db], y_sh.at[didx[mb]],
                                      sem_s.at[db]).wait()

            def scale(mb, db):
                data_b = data[db]

                def grp(g2, carry2):
                    cvec16 = coef_v[mb, pl.ds(g2 * L, L)]
                    for t in range(L):
                        tb = g2 * L + t
                        bc = cvec16.at[jnp.full((L,), t, i32)].get(
                            mode="promise_in_bounds")
                        for j in range(half // L):
                            sl = pl.ds(j * L, L)
                            data_b[tb, sl] = data_b[tb, sl] * bc
                    return carry2
                lax.fori_loop(0, 128 // L, grp, 0)

            meta_start(0, 0)
            meta_start(1, 1)
            meta_wait(0, 0)
            gather_start(0, 0)

            ngrp = spt // 4

            def group(g, carry):
                for u in range(4):
                    k = g * 4 + u
                    db, odb = u % 2, (u + 1) % 2
                    mb, mb1, mb2 = u, (u + 1) % 4, (u + 2) % 4
                    gather_wait(mb, db)
                    if u > 0:
                        scatter_wait((u - 1) % 4, odb)
                    else:
                        @pl.when(g > 0)
                        def _():
                            scatter_wait(3, odb)
                    if u < 3:
                        meta_wait(k + 1, mb1)
                        gather_start(mb1, odb)
                    else:
                        @pl.when(g < ngrp - 1)
                        def _():
                            meta_wait(k + 1, mb1)
                            gather_start(mb1, odb)
                    scale(mb, db)
                    scatter_start(mb, db)
                    if u < 2:
                        meta_start(k + 2, mb2)
                    else:
                        @pl.when(g < ngrp - 1)
                        def _():
                            meta_start(k + 2, mb2)
                return carry
            lax.fori_loop(0, ngrp, group, 0)

            scatter_wait((spt - 1) % 4, (spt - 1) % 2)

            plsc.subcore_barrier()
            pltpu.sync_copy(
                y_sh.at[pl.ds(sid * rows_t, rows_t)],
                yout_ref.at[pl.ds(cid * n_rows + sid * rows_t, rows_t)])

    ycat = _msg_kernel(xwflat, idxd, idxs, coef)
    ya = ycat[:n]
    yb = ycat[n_rows:n_rows + n]

    nb = n // bx
    out = pl.pallas_call(
        _fin_body,
        grid=(nb,),
        in_specs=[
            pl.BlockSpec((bx, half), lambda i: (i, 0)),
            pl.BlockSpec((bx, half), lambda i: (i, 0)),
            pl.BlockSpec((bx, half), lambda i: (i, 0)),
            pl.BlockSpec((bx, half), lambda i: (i, 0)),
            pl.BlockSpec((1, 1, bx), lambda i: (i, 0, 0)),
        ],
        out_specs=pl.BlockSpec((bx, d), lambda i: (i, 0)),
        out_shape=jax.ShapeDtypeStruct((n, d), f32),
    )(ya, yb, xw2[0], xw2[1], deg.reshape(nb, 1, bx))
    return out
